# baseline (speedup 1.0000x reference)
import jax
import jax.numpy as jnp
from jax.experimental import pallas as pl
from jax.experimental.pallas import tpu as pltpu

TOPK = 8
N_EXPERTS = 64
HIDDEN = 2048
BLOCK_ROWS = 2048


def _gate_kernel(x_ref, w_hbm_ref, idx_ref, wgt_ref, w_vmem, w_sem):
    @pl.when(pl.program_id(0) == 0)
    def _load_weight():
        cp = pltpu.make_async_copy(w_hbm_ref, w_vmem, w_sem)
        cp.start()
        cp.wait()

    logits = jax.lax.dot_general(
        x_ref[...], w_vmem[...], (((1,), (0,)), ((), ())),
        preferred_element_type=jnp.float32,
    )

    iota = jax.lax.broadcasted_iota(jnp.int32, logits.shape, 1).astype(jnp.float32)
    vals = []
    idxs = []
    work = logits
    for _ in range(TOPK):
        v = jnp.max(work, axis=-1, keepdims=True)
        i = jnp.min(
            jnp.where(work == v, iota, jnp.float32(N_EXPERTS)),
            axis=-1,
            keepdims=True,
        )
        vals.append(v)
        idxs.append(i)
        work = jnp.where(iota == i, -jnp.inf, work)

    topv = jnp.concatenate(vals, axis=-1)
    topi = jnp.concatenate(idxs, axis=-1)
    e = jnp.exp(topv - vals[0])
    wgt_ref[...] = e / jnp.sum(e, axis=-1, keepdims=True)
    idx_ref[...] = topi.astype(jnp.int32)


def _gate(x, weight_t):
    n = x.shape[0]
    grid = (n // BLOCK_ROWS,)
    idx, wgt = pl.pallas_call(
        _gate_kernel,
        grid=grid,
        in_specs=[
            pl.BlockSpec((BLOCK_ROWS, HIDDEN), lambda i: (i, 0)),
            pl.BlockSpec(memory_space=pl.ANY),
        ],
        out_specs=[
            pl.BlockSpec((BLOCK_ROWS, TOPK), lambda i: (i, 0)),
            pl.BlockSpec((BLOCK_ROWS, TOPK), lambda i: (i, 0)),
        ],
        out_shape=[
            jax.ShapeDtypeStruct((n, TOPK), jnp.int32),
            jax.ShapeDtypeStruct((n, TOPK), jnp.float32),
        ],
        scratch_shapes=[
            pltpu.VMEM((HIDDEN, N_EXPERTS), jnp.float32),
            pltpu.SemaphoreType.DMA,
        ],
    )(x, weight_t)
    return idx, wgt


def kernel(hidden_states, weight):
    b, s, h = hidden_states.shape
    x = hidden_states.reshape(-1, h)
    topk_idx, topk_weight = _gate(x, weight.T)
    aux_loss = jnp.array(0.0, dtype=jnp.float32)
    return (topk_idx, topk_weight, aux_loss)


# speedup vs baseline: 1.0567x; 1.0567x over previous
import jax
import jax.numpy as jnp
from jax.experimental import pallas as pl
from jax.experimental.pallas import tpu as pltpu

TOPK = 8
N_EXPERTS = 64
HIDDEN = 2048
BLOCK_ROWS = 2048


def _gate_kernel(x1_ref, x2_ref, w1_ref, w2_ref, idx_ref, wgt_ref):
    logits = jax.lax.dot_general(
        x1_ref[...], w1_ref[...], (((1,), (0,)), ((), ())),
        preferred_element_type=jnp.float32,
    ) + jax.lax.dot_general(
        x2_ref[...], w2_ref[...], (((1,), (0,)), ((), ())),
        preferred_element_type=jnp.float32,
    )

    iota = jax.lax.broadcasted_iota(jnp.int32, logits.shape, 1).astype(jnp.float32)
    vals = []
    idxs = []
    work = logits
    for _ in range(TOPK):
        v = jnp.max(work, axis=-1, keepdims=True)
        i = jnp.min(
            jnp.where(work == v, iota, jnp.float32(N_EXPERTS)),
            axis=-1,
            keepdims=True,
        )
        vals.append(v)
        idxs.append(i)
        work = jnp.where(iota == i, -jnp.inf, work)

    topv = jnp.concatenate(vals, axis=-1)
    topi = jnp.concatenate(idxs, axis=-1)
    e = jnp.exp(topv - vals[0])
    wgt_ref[...] = e / jnp.sum(e, axis=-1, keepdims=True)
    idx_ref[...] = topi.astype(jnp.int32)


def _gate(x, weight_t):
    n = x.shape[0]
    h2 = HIDDEN // 2
    grid = (n // BLOCK_ROWS,)
    idx, wgt = pl.pallas_call(
        _gate_kernel,
        grid=grid,
        in_specs=[
            pl.BlockSpec((BLOCK_ROWS, h2), lambda i: (i, 0)),
            pl.BlockSpec((BLOCK_ROWS, h2), lambda i: (i, 1)),
            pl.BlockSpec((h2, N_EXPERTS), lambda i: (0, 0)),
            pl.BlockSpec((h2, N_EXPERTS), lambda i: (1, 0)),
        ],
        out_specs=[
            pl.BlockSpec((BLOCK_ROWS, TOPK), lambda i: (i, 0)),
            pl.BlockSpec((BLOCK_ROWS, TOPK), lambda i: (i, 0)),
        ],
        out_shape=[
            jax.ShapeDtypeStruct((n, TOPK), jnp.int32),
            jax.ShapeDtypeStruct((n, TOPK), jnp.float32),
        ],
        compiler_params=pltpu.CompilerParams(
            dimension_semantics=("parallel",),
        ),
    )(x, x, weight_t, weight_t)
    return idx, wgt


def kernel(hidden_states, weight):
    b, s, h = hidden_states.shape
    x = hidden_states.reshape(-1, h)
    topk_idx, topk_weight = _gate(x, weight.T)
    aux_loss = jnp.array(0.0, dtype=jnp.float32)
    return (topk_idx, topk_weight, aux_loss)
